# Initial kernel scaffold; baseline (speedup 1.0000x reference)
import jax
import jax.numpy as jnp
from jax.experimental import pallas as pl


def kernel(x, W0, b0, W1, b1, W2, b2, gamma2, beta2, W3, b3, W4, b4, Wf, bf):
    raise NotImplementedError("write your pallas kernel here")


# speedup vs baseline: 6.1794x; 6.1794x over previous
import functools

import jax
import jax.numpy as jnp
from jax import lax
from jax.experimental import pallas as pl
from jax.experimental.pallas import tpu as pltpu
from jax.experimental.pallas import tpu_sc as plsc

F32 = jnp.float32
I32 = jnp.int32
BF16 = jnp.bfloat16
KNN = 40
KPAD = 48
NPT = 2048
NBATCH = 8
BN = NBATCH * NPT
RB = 128
RC = 128
NCH = NPT // 16
NCV = NCH // 16
SELU_A = 1.6732632423543772848170429916717
SELU_S = 1.0507009873554804934193349852946
_GDN = lax.GatherDimensionNumbers(offset_dims=(), collapsed_slice_dims=(0,),
                                  start_index_map=(0,))
BIGI = 1 << 20


def _selu(x):
    return jnp.where(x > 0, SELU_S * x, (SELU_S * SELU_A) * (jnp.exp(x) - 1.0))


def _shuf(x, idx):
    return lax.gather(x, idx[:, None], _GDN, (1,),
                      mode=lax.GatherScatterMode.PROMISE_IN_BOUNDS)


def _splat_min(x, iota):
    for sh in (8, 4, 2, 1):
        x = jnp.minimum(x, _shuf(x, iota ^ sh))
    return x


def _sc1(x):
    return jnp.squeeze(lax.slice(x, (0,), (1,)))


def _sqrow_body(x_ref, o_ref):
    x = x_ref[0]
    o_ref[0] = jnp.sum(x * x, axis=0, keepdims=True)


def _sqrow(x_cn):
    bsz, cin, n = x_cn.shape
    return pl.pallas_call(
        _sqrow_body,
        grid=(bsz,),
        in_specs=[pl.BlockSpec((1, cin, n), lambda b: (b, 0, 0))],
        out_specs=pl.BlockSpec((1, 1, n), lambda b: (b, 0, 0)),
        out_shape=jax.ShapeDtypeStruct((bsz, 1, n), F32),
    )(x_cn)


def _dist_body(xb_ref, xf_ref, sq_ref, dist_ref):
    xb = xb_ref[0]
    xb16 = xb.astype(BF16)
    xf16 = xf_ref[0].astype(BF16)
    g = lax.dot_general(xb16, xf16, (((1,), (1,)), ((), ())),
                        preferred_element_type=F32)
    sqb = jnp.sum(xb * xb, axis=1, keepdims=True)
    dist_ref[0] = (-2.0 * g + sqb) + sq_ref[0]


def _dist(xT, sq_row):
    bsz, n, cin = xT.shape
    return pl.pallas_call(
        _dist_body,
        grid=(bsz, n // RB),
        in_specs=[
            pl.BlockSpec((1, RB, cin), lambda b, i: (b, i, 0)),
            pl.BlockSpec((1, n, cin), lambda b, i: (b, 0, 0)),
            pl.BlockSpec((1, 1, n), lambda b, i: (b, 0, 0)),
        ],
        out_specs=pl.BlockSpec((1, RB, n), lambda b, i: (b, i, 0)),
        out_shape=jax.ShapeDtypeStruct((bsz, n, n), F32),
    )(xT, xT, sq_row)


def _make_sc_knn(cin):
    info = plsc.get_sparse_core_info()
    ntiles = info.num_cores * info.num_subcores
    rpt = BN // ntiles
    tiles_per_batch = NPT // rpt

    @functools.partial(
        pl.kernel,
        out_type=jax.ShapeDtypeStruct((BN, KNN, cin), F32),
        mesh=plsc.VectorSubcoreMesh(core_axis_name="c", subcore_axis_name="s"),
        compiler_params=pltpu.CompilerParams(use_tc_tiling_on_sc=False),
        scratch_types=[
            pltpu.VMEM((NPT,), F32),
            pltpu.VMEM((KPAD,), I32),
            pltpu.VMEM((KNN, cin), F32),
            pltpu.SemaphoreType.DMA,
        ],
    )
    def sc_kernel(dist_h, xt_h, xj_h, drow, idxv, gat, sem):
        wid = lax.axis_index("s") * info.num_cores + lax.axis_index("c")
        batch = wid // tiles_per_batch
        base = wid * rpt
        gbase = batch * NPT
        iota = lax.iota(I32, 16)
        INF = jnp.float32(3.0e38)

        def row_body(r, _):
            pltpu.sync_copy(dist_h.at[base + r], drow)

            def p1(i, cms):
                v = drow[pl.ds(i * 16, 16)]
                mn = _splat_min(v, iota)
                g_t = i // 16
                l_t = i % 16
                return tuple(
                    jnp.where(iota == jnp.where(g_t == g, l_t, -1), mn, cms[g])
                    for g in range(NCV))

            cms = lax.fori_loop(0, NCH, p1,
                                tuple(jnp.full((16,), INF, F32)
                                      for _ in range(NCV)))

            def p2(t, carry):
                cms, idxs = carry
                gm = cms[0]
                for g in range(1, NCV):
                    gm = jnp.minimum(gm, cms[g])
                gm = _splat_min(gm, iota)
                cid = jnp.full((16,), BIGI, I32)
                for g in range(NCV):
                    cid = jnp.minimum(
                        cid, jnp.where(cms[g] == gm, iota + g * 16, BIGI))
                cid = _splat_min(cid, iota)
                cstar = _sc1(cid)
                v = drow[pl.ds(cstar * 16, 16)]
                lid = jnp.where(v == gm, iota, BIGI)
                lid = _splat_min(lid, iota)
                lstar = _sc1(lid)
                jstar = cstar * 16 + lstar + gbase
                tv = t // 16
                tl = t % 16
                idxs = tuple(
                    jnp.where(iota == jnp.where(tv == q, tl, -1), jstar, idxs[q])
                    for q in range(3))
                v2 = jnp.where(iota == lstar, INF, v)
                drow[pl.ds(cstar * 16, 16)] = v2
                nm = _splat_min(v2, iota)
                gstar = cstar // 16
                lpos = cstar % 16
                cms = tuple(
                    jnp.where(iota == jnp.where(gstar == g, lpos, -1), nm, cms[g])
                    for g in range(NCV))
                return cms, idxs

            idx0 = tuple(jnp.zeros((16,), I32) for _ in range(3))
            _, idxs = lax.fori_loop(0, KNN, p2, (cms, idx0))
            for q in range(3):
                idxv[pl.ds(q * 16, 16)] = idxs[q]
            pltpu.async_copy(xt_h.at[idxv.at[pl.ds(0, KNN)]], gat, sem).wait()
            pltpu.sync_copy(gat, xj_h.at[base + r])
            return 0

        lax.fori_loop(0, rpt, row_body, 0)

    return sc_kernel


def _conv_body(nbn, xi_ref, xj_ref, w_ref, b_ref, o_ref, *bn_refs):
    xi = xi_ref[0]
    xj = xj_ref[0]
    rc, cin = xi.shape
    xi3 = jnp.broadcast_to(xi[:, None, :], (rc, KNN, cin))
    f1 = xi3.astype(BF16).reshape(rc * KNN, cin)
    f2 = (xj - xi3).astype(BF16).reshape(rc * KNN, cin)
    feat = jnp.concatenate([f1, f2], axis=1)
    e = lax.dot_general(feat, w_ref[...], (((1,), (0,)), ((), ())),
                        preferred_element_type=F32) + b_ref[...]
    cout = e.shape[1]
    m = jnp.max(e.reshape(rc, KNN, cout), axis=1)
    if nbn:
        ps_ref, pq_ref = bn_refs
        ps_ref[0] = jnp.sum(e, axis=0, keepdims=True)
        pq_ref[0] = jnp.sum(e * e, axis=0, keepdims=True)
        o_ref[0] = m
    else:
        o_ref[0] = _selu(m)


def _conv(xT, xj4, w2c, b2, bn):
    bsz, n, cin = xT.shape
    cout = w2c.shape[1]
    nblk = n // RC
    out_shape = [jax.ShapeDtypeStruct((bsz, n, cout), F32)]
    out_specs = [pl.BlockSpec((1, RC, cout), lambda b, i: (b, i, 0))]
    if bn:
        out_shape += [jax.ShapeDtypeStruct((bsz * nblk, 1, cout), F32)] * 2
        out_specs += [pl.BlockSpec((1, 1, cout),
                                   lambda b, i: (b * nblk + i, 0, 0))] * 2
    res = pl.pallas_call(
        functools.partial(_conv_body, bn),
        grid=(bsz, nblk),
        in_specs=[
            pl.BlockSpec((1, RC, cin), lambda b, i: (b, i, 0)),
            pl.BlockSpec((1, RC, KNN, cin), lambda b, i: (b, i, 0, 0)),
            pl.BlockSpec((2 * cin, cout), lambda b, i: (0, 0)),
            pl.BlockSpec((1, cout), lambda b, i: (0, 0)),
        ],
        out_specs=tuple(out_specs),
        out_shape=tuple(out_shape),
    )(xT, xj4, w2c, b2)
    return res if bn else res[0]


def _bn_body(m_ref, ps_ref, pq_ref, g_ref, be_ref, o_ref):
    cnt = jnp.float32(BN * KNN)
    mean = jnp.sum(ps_ref[...], axis=0, keepdims=True) / cnt
    esq = jnp.sum(pq_ref[...], axis=0, keepdims=True) / cnt
    var = esq - mean * mean
    scale = g_ref[...] * lax.rsqrt(var + 1e-5)
    o_ref[...] = _selu((m_ref[...] - mean) * scale + be_ref[...])


def _bn(m2, ps, pq, gamma, beta):
    rows, cout = m2.shape
    nb = ps.shape[0]
    return pl.pallas_call(
        _bn_body,
        grid=(),
        in_specs=[
            pl.BlockSpec((rows, cout), lambda: (0, 0)),
            pl.BlockSpec((nb, cout), lambda: (0, 0)),
            pl.BlockSpec((nb, cout), lambda: (0, 0)),
            pl.BlockSpec((1, cout), lambda: (0, 0)),
            pl.BlockSpec((1, cout), lambda: (0, 0)),
        ],
        out_specs=pl.BlockSpec((rows, cout), lambda: (0, 0)),
        out_shape=jax.ShapeDtypeStruct((rows, cout), F32),
    )(m2, ps, pq, gamma.reshape(1, cout), beta.reshape(1, cout))


def _final_body(cat_ref, w_ref, b_ref, o_ref):
    o_ref[0] = _selu(
        lax.dot_general(w_ref[...].astype(BF16), cat_ref[0].astype(BF16),
                        (((1,), (1,)), ((), ())),
                        preferred_element_type=F32) + b_ref[...])


def _final(cat3, wf, b2col):
    bsz, n, twoc = cat3.shape
    emb = wf.shape[0]
    rf = 512
    return pl.pallas_call(
        _final_body,
        grid=(bsz, n // rf),
        in_specs=[
            pl.BlockSpec((1, rf, twoc), lambda b, i: (b, i, 0)),
            pl.BlockSpec((emb, twoc), lambda b, i: (0, 0)),
            pl.BlockSpec((emb, 1), lambda b, i: (0, 0)),
        ],
        out_specs=pl.BlockSpec((1, emb, rf), lambda b, i: (b, 0, i)),
        out_shape=jax.ShapeDtypeStruct((bsz, emb, n), F32),
    )(cat3, wf, b2col)


def _edge_layer(xT, W, b, gamma=None, beta=None):
    bsz, n, cin = xT.shape
    cout = W.shape[0]
    x_cn = jnp.transpose(xT, (0, 2, 1))
    sq_row = _sqrow(x_cn)
    dist = _dist(xT, sq_row)
    cpad = max(cin, 8)
    xg = xT if cpad == cin else jnp.pad(xT, ((0, 0), (0, 0), (0, cpad - cin)))
    sc = _make_sc_knn(cpad)
    xj = sc(dist.reshape(BN, n), xg.reshape(BN, cpad))
    xj4 = xj.reshape(bsz, n, KNN, cpad)[..., :cin]
    w2c = W.T.astype(BF16)
    if gamma is None:
        out = _conv(xT, xj4, w2c, b.reshape(1, cout), False)
    else:
        m, ps, pq = _conv(xT, xj4, w2c, b.reshape(1, cout), True)
        out = _bn(m.reshape(BN, cout), ps.reshape(-1, cout),
                  pq.reshape(-1, cout), gamma, beta).reshape(bsz, n, cout)
    return out


def kernel(x, W0, b0, W1, b1, W2, b2, gamma2, beta2, W3, b3, W4, b4, Wf, bf):
    xT = jnp.transpose(x[..., 0], (0, 2, 1))
    outs = []
    h = _edge_layer(xT, W0, b0); outs.append(h)
    h = _edge_layer(h, W1, b1); outs.append(h)
    h = _edge_layer(h, W2, b2, gamma2, beta2); outs.append(h)
    h = _edge_layer(h, W3, b3); outs.append(h)
    h = _edge_layer(h, W4, b4); outs.append(h)
    cat = jnp.concatenate(outs, axis=-1)
    cat2 = jnp.concatenate([cat, jnp.zeros_like(cat)], axis=-1)
    out = _final(cat2, Wf, bf.reshape(-1, 1))
    return out[..., None]


# speedup vs baseline: 7.1345x; 1.1546x over previous
import functools

import jax
import jax.numpy as jnp
from jax import lax
from jax.experimental import pallas as pl
from jax.experimental.pallas import tpu as pltpu
from jax.experimental.pallas import tpu_sc as plsc

F32 = jnp.float32
I32 = jnp.int32
BF16 = jnp.bfloat16
KNN = 40
KPAD = 48
NPT = 2048
NBATCH = 8
BN = NBATCH * NPT
RB = 128
RC = 128
NCH = NPT // 16
NCV = NCH // 16
SELU_A = 1.6732632423543772848170429916717
SELU_S = 1.0507009873554804934193349852946
_GDN = lax.GatherDimensionNumbers(offset_dims=(), collapsed_slice_dims=(0,),
                                  start_index_map=(0,))
BIGI = 1 << 20


def _selu(x):
    return jnp.where(x > 0, SELU_S * x, (SELU_S * SELU_A) * (jnp.exp(x) - 1.0))


def _shuf(x, idx):
    return lax.gather(x, idx[:, None], _GDN, (1,),
                      mode=lax.GatherScatterMode.PROMISE_IN_BOUNDS)


def _splat_min(x, iota):
    for sh in (8, 4, 2, 1):
        x = jnp.minimum(x, _shuf(x, iota ^ sh))
    return x


def _sc1(x):
    return jnp.squeeze(lax.slice(x, (0,), (1,)))


def _sqrow_body(x_ref, o_ref):
    x = x_ref[0]
    o_ref[0] = jnp.sum(x * x, axis=0, keepdims=True)


def _sqrow(x_cn):
    bsz, cin, n = x_cn.shape
    return pl.pallas_call(
        _sqrow_body,
        grid=(bsz,),
        in_specs=[pl.BlockSpec((1, cin, n), lambda b: (b, 0, 0))],
        out_specs=pl.BlockSpec((1, 1, n), lambda b: (b, 0, 0)),
        out_shape=jax.ShapeDtypeStruct((bsz, 1, n), F32),
    )(x_cn)


def _dist_body(xb_ref, xf_ref, sq_ref, dist_ref):
    xb = xb_ref[0]
    xb16 = xb.astype(BF16)
    xf16 = xf_ref[0].astype(BF16)
    g = lax.dot_general(xb16, xf16, (((1,), (1,)), ((), ())),
                        preferred_element_type=F32)
    sqb = jnp.sum(xb * xb, axis=1, keepdims=True)
    dist_ref[0] = (-2.0 * g + sqb) + sq_ref[0]


def _dist(xT, sq_row):
    bsz, n, cin = xT.shape
    return pl.pallas_call(
        _dist_body,
        grid=(bsz, n // RB),
        in_specs=[
            pl.BlockSpec((1, RB, cin), lambda b, i: (b, i, 0)),
            pl.BlockSpec((1, n, cin), lambda b, i: (b, 0, 0)),
            pl.BlockSpec((1, 1, n), lambda b, i: (b, 0, 0)),
        ],
        out_specs=pl.BlockSpec((1, RB, n), lambda b, i: (b, i, 0)),
        out_shape=jax.ShapeDtypeStruct((bsz, n, n), F32),
    )(xT, xT, sq_row)


def _make_sc_knn(cin):
    info = plsc.get_sparse_core_info()
    ntiles = info.num_cores * info.num_subcores
    rpt = BN // ntiles
    tiles_per_batch = NPT // rpt

    @functools.partial(
        pl.kernel,
        out_type=jax.ShapeDtypeStruct((BN, KNN, cin), F32),
        mesh=plsc.VectorSubcoreMesh(core_axis_name="c", subcore_axis_name="s"),
        compiler_params=pltpu.CompilerParams(use_tc_tiling_on_sc=False),
        scratch_types=[
            pltpu.VMEM((NPT,), F32),
            pltpu.VMEM((KPAD,), I32),
            pltpu.VMEM((KNN, cin), F32),
            pltpu.SemaphoreType.DMA,
        ],
    )
    def sc_kernel(dist_h, xt_h, xj_h, drow, idxv, gat, sem):
        wid = lax.axis_index("s") * info.num_cores + lax.axis_index("c")
        batch = wid // tiles_per_batch
        base = wid * rpt
        gbase = batch * NPT
        iota = lax.iota(I32, 16)
        INF = jnp.float32(3.0e38)

        def row_body(r, _):
            pltpu.sync_copy(dist_h.at[base + r], drow)

            cms = []
            for g in range(NCV):
                cm = jnp.full((16,), INF, F32)
                for l_t in range(16):
                    v = drow[pl.ds((g * 16 + l_t) * 16, 16)]
                    mn = _splat_min(v, iota)
                    cm = jnp.where(iota == l_t, mn, cm)
                cms.append(cm)
            cms = tuple(cms)

            def p2(t, carry):
                cms, idxs = carry
                gm = cms[0]
                for g in range(1, NCV):
                    gm = jnp.minimum(gm, cms[g])
                gm = _splat_min(gm, iota)
                cid = jnp.full((16,), BIGI, I32)
                for g in range(NCV):
                    cid = jnp.minimum(
                        cid, jnp.where(cms[g] == gm, iota + g * 16, BIGI))
                cid = _splat_min(cid, iota)
                cstar = _sc1(cid)
                v = drow[pl.ds(cstar * 16, 16)]
                lid = jnp.where(v == gm, iota, BIGI)
                lid = _splat_min(lid, iota)
                lstar = _sc1(lid)
                jstar = cstar * 16 + lstar + gbase
                tv = t // 16
                tl = t % 16
                idxs = tuple(
                    jnp.where(iota == jnp.where(tv == q, tl, -1), jstar, idxs[q])
                    for q in range(3))
                v2 = jnp.where(iota == lstar, INF, v)
                drow[pl.ds(cstar * 16, 16)] = v2
                nm = _splat_min(v2, iota)
                gstar = cstar // 16
                lpos = cstar % 16
                cms = tuple(
                    jnp.where(iota == jnp.where(gstar == g, lpos, -1), nm, cms[g])
                    for g in range(NCV))
                return cms, idxs

            idx0 = tuple(jnp.zeros((16,), I32) for _ in range(3))
            _, idxs = lax.fori_loop(0, KNN, p2, (cms, idx0))
            for q in range(3):
                idxv[pl.ds(q * 16, 16)] = idxs[q]
            pltpu.async_copy(xt_h.at[idxv.at[pl.ds(0, KNN)]], gat, sem).wait()
            pltpu.sync_copy(gat, xj_h.at[base + r])
            return 0

        lax.fori_loop(0, rpt, row_body, 0)

    return sc_kernel


def _conv_body(nbn, xi_ref, xj_ref, w_ref, b_ref, o_ref, *bn_refs):
    xi = xi_ref[0]
    xj = xj_ref[0]
    rc, cin = xi.shape
    xi3 = jnp.broadcast_to(xi[:, None, :], (rc, KNN, cin))
    f1 = xi3.astype(BF16).reshape(rc * KNN, cin)
    f2 = (xj - xi3).astype(BF16).reshape(rc * KNN, cin)
    feat = jnp.concatenate([f1, f2], axis=1)
    e = lax.dot_general(feat, w_ref[...], (((1,), (0,)), ((), ())),
                        preferred_element_type=F32) + b_ref[...]
    cout = e.shape[1]
    m = jnp.max(e.reshape(rc, KNN, cout), axis=1)
    if nbn:
        ps_ref, pq_ref = bn_refs
        ps_ref[0] = jnp.sum(e, axis=0, keepdims=True)
        pq_ref[0] = jnp.sum(e * e, axis=0, keepdims=True)
        o_ref[0] = m
    else:
        o_ref[0] = _selu(m)


def _conv(xT, xj4, w2c, b2, bn):
    bsz, n, cin = xT.shape
    cout = w2c.shape[1]
    nblk = n // RC
    out_shape = [jax.ShapeDtypeStruct((bsz, n, cout), F32)]
    out_specs = [pl.BlockSpec((1, RC, cout), lambda b, i: (b, i, 0))]
    if bn:
        out_shape += [jax.ShapeDtypeStruct((bsz * nblk, 1, cout), F32)] * 2
        out_specs += [pl.BlockSpec((1, 1, cout),
                                   lambda b, i: (b * nblk + i, 0, 0))] * 2
    res = pl.pallas_call(
        functools.partial(_conv_body, bn),
        grid=(bsz, nblk),
        in_specs=[
            pl.BlockSpec((1, RC, cin), lambda b, i: (b, i, 0)),
            pl.BlockSpec((1, RC, KNN, cin), lambda b, i: (b, i, 0, 0)),
            pl.BlockSpec((2 * cin, cout), lambda b, i: (0, 0)),
            pl.BlockSpec((1, cout), lambda b, i: (0, 0)),
        ],
        out_specs=tuple(out_specs),
        out_shape=tuple(out_shape),
    )(xT, xj4, w2c, b2)
    return res if bn else res[0]


def _bn_body(m_ref, ps_ref, pq_ref, g_ref, be_ref, o_ref):
    cnt = jnp.float32(BN * KNN)
    mean = jnp.sum(ps_ref[...], axis=0, keepdims=True) / cnt
    esq = jnp.sum(pq_ref[...], axis=0, keepdims=True) / cnt
    var = esq - mean * mean
    scale = g_ref[...] * lax.rsqrt(var + 1e-5)
    o_ref[...] = _selu((m_ref[...] - mean) * scale + be_ref[...])


def _bn(m2, ps, pq, gamma, beta):
    rows, cout = m2.shape
    nb = ps.shape[0]
    return pl.pallas_call(
        _bn_body,
        grid=(),
        in_specs=[
            pl.BlockSpec((rows, cout), lambda: (0, 0)),
            pl.BlockSpec((nb, cout), lambda: (0, 0)),
            pl.BlockSpec((nb, cout), lambda: (0, 0)),
            pl.BlockSpec((1, cout), lambda: (0, 0)),
            pl.BlockSpec((1, cout), lambda: (0, 0)),
        ],
        out_specs=pl.BlockSpec((rows, cout), lambda: (0, 0)),
        out_shape=jax.ShapeDtypeStruct((rows, cout), F32),
    )(m2, ps, pq, gamma.reshape(1, cout), beta.reshape(1, cout))


def _final_body(cat_ref, w_ref, b_ref, o_ref):
    o_ref[0] = _selu(
        lax.dot_general(w_ref[...].astype(BF16), cat_ref[0].astype(BF16),
                        (((1,), (1,)), ((), ())),
                        preferred_element_type=F32) + b_ref[...])


def _final(cat3, wf, b2col):
    bsz, n, twoc = cat3.shape
    emb = wf.shape[0]
    rf = 512
    return pl.pallas_call(
        _final_body,
        grid=(bsz, n // rf),
        in_specs=[
            pl.BlockSpec((1, rf, twoc), lambda b, i: (b, i, 0)),
            pl.BlockSpec((emb, twoc), lambda b, i: (0, 0)),
            pl.BlockSpec((emb, 1), lambda b, i: (0, 0)),
        ],
        out_specs=pl.BlockSpec((1, emb, rf), lambda b, i: (b, 0, i)),
        out_shape=jax.ShapeDtypeStruct((bsz, emb, n), F32),
    )(cat3, wf, b2col)


def _edge_layer(xT, W, b, gamma=None, beta=None):
    bsz, n, cin = xT.shape
    cout = W.shape[0]
    x_cn = jnp.transpose(xT, (0, 2, 1))
    sq_row = _sqrow(x_cn)
    dist = _dist(xT, sq_row)
    cpad = max(cin, 8)
    xg = xT if cpad == cin else jnp.pad(xT, ((0, 0), (0, 0), (0, cpad - cin)))
    sc = _make_sc_knn(cpad)
    xj = sc(dist.reshape(BN, n), xg.reshape(BN, cpad))
    xj4 = xj.reshape(bsz, n, KNN, cpad)[..., :cin]
    w2c = W.T.astype(BF16)
    if gamma is None:
        out = _conv(xT, xj4, w2c, b.reshape(1, cout), False)
    else:
        m, ps, pq = _conv(xT, xj4, w2c, b.reshape(1, cout), True)
        out = _bn(m.reshape(BN, cout), ps.reshape(-1, cout),
                  pq.reshape(-1, cout), gamma, beta).reshape(bsz, n, cout)
    return out


def kernel(x, W0, b0, W1, b1, W2, b2, gamma2, beta2, W3, b3, W4, b4, Wf, bf):
    xT = jnp.transpose(x[..., 0], (0, 2, 1))
    outs = []
    h = _edge_layer(xT, W0, b0); outs.append(h)
    h = _edge_layer(h, W1, b1); outs.append(h)
    h = _edge_layer(h, W2, b2, gamma2, beta2); outs.append(h)
    h = _edge_layer(h, W3, b3); outs.append(h)
    h = _edge_layer(h, W4, b4); outs.append(h)
    cat = jnp.concatenate(outs, axis=-1)
    cat2 = jnp.concatenate([cat, jnp.zeros_like(cat)], axis=-1)
    out = _final(cat2, Wf, bf.reshape(-1, 1))
    return out[..., None]
